# Initial kernel scaffold; baseline (speedup 1.0000x reference)
import jax
import jax.numpy as jnp
from jax.experimental import pallas as pl


def kernel(features, pid_labels, large_batch_queue, tail):
    raise NotImplementedError("write your pallas kernel here")


# speedup vs baseline: 4.0412x; 4.0412x over previous
import functools

import jax
import jax.numpy as jnp
from jax import lax
from jax.experimental import pallas as pl
from jax.experimental.pallas import tpu as pltpu
from jax.experimental.pallas import tpu_sc as plsc

NUM_CLASSES = 37
NUM_INSTANCE = 64
FEAT = 256
N_ROWS = 65536
LANES = 16

NUM_CORES = 2
NUM_SUBCORES = 16
NW = NUM_CORES * NUM_SUBCORES
ROWS_PER_W = N_ROWS // NW
CHUNK = 128
NCHUNK = ROWS_PER_W // CHUNK
CPAD = 40


def _sc_body(feat_hbm, lab_hbm, sums_out, cnts_out,
             rows_v, labels_v, ones_v, sums_acc, cnts_acc):
    cid = lax.axis_index("c")
    sid = lax.axis_index("s")
    wid = cid * NUM_SUBCORES + sid

    zeros16 = jnp.zeros((LANES,), jnp.float32)
    ones16 = jnp.ones((LANES,), jnp.float32)

    @pl.when(sid == 0)
    def _():
        for i in range(CPAD):
            for j in range(FEAT // LANES):
                rows_v[i, pl.ds(j * LANES, LANES)] = zeros16
            ones_v[i, :] = zeros16
        pltpu.sync_copy(rows_v.at[pl.ds(0, CPAD), :], sums_acc)
        pltpu.sync_copy(ones_v.at[pl.ds(0, CPAD), :], cnts_acc)

    for i in range(CHUNK):
        ones_v[i, :] = ones16

    pltpu.sync_copy(lab_hbm.at[wid], labels_v)
    plsc.subcore_barrier()

    base = wid * ROWS_PER_W
    for k in range(NCHUNK):
        pltpu.sync_copy(feat_hbm.at[pl.ds(base + k * CHUNK, CHUNK), :], rows_v)
        pltpu.sync_copy(rows_v, sums_acc.at[labels_v.at[k]], add=True)
        pltpu.sync_copy(ones_v, cnts_acc.at[labels_v.at[k]], add=True)

    plsc.subcore_barrier()

    @pl.when(sid == 0)
    def _():
        pltpu.sync_copy(sums_acc, sums_out.at[cid])
        pltpu.sync_copy(cnts_acc, cnts_out.at[cid])


_sc_accum = functools.partial(
    pl.kernel,
    out_type=(
        jax.ShapeDtypeStruct((NUM_CORES, CPAD, FEAT), jnp.float32),
        jax.ShapeDtypeStruct((NUM_CORES, CPAD, LANES), jnp.float32),
    ),
    mesh=plsc.VectorSubcoreMesh(core_axis_name="c", subcore_axis_name="s"),
    scratch_types=[
        pltpu.VMEM((CHUNK, FEAT), jnp.float32),
        pltpu.VMEM((NCHUNK, CHUNK), jnp.int32),
        pltpu.VMEM((CHUNK, LANES), jnp.float32),
        pltpu.VMEM_SHARED((CPAD, FEAT), jnp.float32),
        pltpu.VMEM_SHARED((CPAD, LANES), jnp.float32),
    ],
    compiler_params=pltpu.CompilerParams(use_tc_tiling_on_sc=False),
)(_sc_body)


def _combine_body(p_ref, c_ref, q_ref, t_ref, o_ref):
    sums = p_ref[0, :NUM_CLASSES]
    cnts = c_ref[0, :NUM_CLASSES]
    for w in range(1, NUM_CORES):
        sums = sums + p_ref[w, :NUM_CLASSES]
        cnts = cnts + c_ref[w, :NUM_CLASSES]
    cnt = cnts[:, 0:1]
    mean = sums / jnp.maximum(cnt, 1.0)
    present = cnt > 0.0
    tail = t_ref[...]
    for j in range(NUM_INSTANCE):
        hit = (tail == j) & present
        o_ref[:, j, :] = jnp.where(hit, mean, q_ref[:, j, :])


def _combine(sums, cnts, queue, tail2d):
    return pl.pallas_call(
        _combine_body,
        out_shape=jax.ShapeDtypeStruct((NUM_CLASSES, NUM_INSTANCE, FEAT),
                                       jnp.float32),
    )(sums, cnts, queue, tail2d)


def kernel(features, pid_labels, large_batch_queue, tail):
    labels_r = pid_labels.reshape(NW, NCHUNK, CHUNK)
    sums, cnts = _sc_accum(features, labels_r)
    return _combine(sums, cnts, large_batch_queue,
                    tail.reshape(NUM_CLASSES, 1))
